# Initial kernel scaffold; baseline (speedup 1.0000x reference)
import jax
import jax.numpy as jnp
from jax.experimental import pallas as pl


def kernel(x, e1, e2, l1_w, l1_b, l2_w, l2_b, d1_w, d1_b, d2_w, d2_b, idx):
    raise NotImplementedError("write your pallas kernel here")


# speedup vs baseline: 2.8725x; 2.8725x over previous
import functools

import jax
import jax.numpy as jnp
from jax.experimental import pallas as pl

ALPHA = 3.0
K = 30


def _adj_body(adj_ref, scored_ref, adjm_ref, rsum_ref):
    adj = adj_ref[...]
    scored = scored_ref[...]
    rb = adj.shape[0]
    lo = jnp.zeros((rb, 1), jnp.float32)
    hi = jnp.full((rb, 1), 1.02, jnp.float32)
    for _ in range(45):
        mid = 0.5 * (lo + hi)
        cnt = jnp.sum(jnp.where(scored > mid, 1.0, 0.0), axis=1,
                      keepdims=True)
        pred = cnt >= K
        lo = jnp.where(pred, mid, lo)
        hi = jnp.where(pred, hi, mid)
    t = hi
    gt = scored > t
    cg = jnp.sum(jnp.where(gt, 1.0, 0.0), axis=1, keepdims=True)
    ties = scored == t
    need = K - cg
    n = adj.shape[1]
    col = jax.lax.broadcasted_iota(jnp.int32, (rb, n), 1)
    clo = jnp.full((rb, 1), -1, jnp.int32)
    chi = jnp.full((rb, 1), n - 1, jnp.int32)
    for _ in range(12):
        cmid = (clo + chi) // 2
        c = jnp.sum(jnp.where(ties & (col <= cmid), 1.0, 0.0), axis=1,
                    keepdims=True)
        pred = c >= need
        chi = jnp.where(pred, cmid, chi)
        clo = jnp.where(pred, clo, cmid)
    sel = gt | (ties & (col <= chi))
    adjm = jnp.where(sel, adj, 0.0)
    adjm_ref[...] = adjm
    rsum_ref[...] = jnp.sum(adjm, axis=1, keepdims=True) + 1.0


def _scale_body(adjmt_ref, normr_ref, normc_ref, ant_ref, *, rblk):
    r0 = pl.program_id(0) * rblk
    blk = adjmt_ref[...]
    nrows, ncols = blk.shape
    rowi = jax.lax.broadcasted_iota(jnp.int32, (nrows, ncols), 0) + r0
    coli = jax.lax.broadcasted_iota(jnp.int32, (nrows, ncols), 1)
    a2t = blk + jnp.where(rowi == coli, 1.0, 0.0)
    ant_ref[...] = normr_ref[...] * a2t * normc_ref[...]


def _prop_body(x_ref, ant_ref, d1w_ref, d1b_ref, d2w_ref, d2b_ref, out_ref,
               *, bs):
    ant = ant_ref[...]
    h1s = []
    for k in range(bs):
        xk = x_ref[k]
        y1 = jnp.dot(d1w_ref[...], xk, preferred_element_type=jnp.float32)
        h1s.append(y1 + d1b_ref[...])
    h1 = jnp.concatenate(h1s, axis=0)
    z1 = jnp.maximum(jnp.dot(h1, ant, preferred_element_type=jnp.float32), 0.0)
    w1 = d1w_ref.shape[0]
    h2s = []
    for k in range(bs):
        z1k = z1[k * w1:(k + 1) * w1]
        y2 = jnp.dot(d2w_ref[...], z1k, preferred_element_type=jnp.float32)
        h2s.append(y2 + d2b_ref[...])
    h2 = jnp.concatenate(h2s, axis=0)
    z2 = jnp.maximum(jnp.dot(h2, ant, preferred_element_type=jnp.float32), 0.0)
    w2 = d2w_ref.shape[0]
    for k in range(bs):
        out_ref[k] = z2[k * w2:(k + 1) * w2]


def kernel(x, e1, e2, l1_w, l1_b, l2_w, l2_b, d1_w, d1_b, d2_w, d2_b, idx):
    n, w = e1.shape
    b = x.shape[0]
    w1 = d1_w.shape[0]
    w2 = d2_w.shape[0]

    e1g = jnp.take(e1, idx, axis=0)
    e2g = jnp.take(e2, idx, axis=0)
    noise = jax.random.uniform(jax.random.key(12345), (n, n),
                               dtype=jnp.float32) * 0.01

    m1 = jnp.tanh(ALPHA * (e1g @ l1_w.T + l1_b))
    m2 = jnp.tanh(ALPHA * (e2g @ l2_w.T + l2_b))
    adj = jax.nn.relu(jnp.tanh(ALPHA * (m1 @ m2.T)))
    scored = adj + noise

    rblk = n // 5 if n % 5 == 0 and (n // 5) % 8 == 0 else n
    grid_r = n // rblk
    adjm, rsum = pl.pallas_call(
        _adj_body,
        grid=(grid_r,),
        in_specs=[
            pl.BlockSpec((rblk, n), lambda i: (i, 0)),
            pl.BlockSpec((rblk, n), lambda i: (i, 0)),
        ],
        out_specs=(pl.BlockSpec((rblk, n), lambda i: (i, 0)),
                   pl.BlockSpec((rblk, 1), lambda i: (i, 0))),
        out_shape=(jax.ShapeDtypeStruct((n, n), jnp.float32),
                   jax.ShapeDtypeStruct((n, 1), jnp.float32)),
    )(adj, scored)

    norm = jax.lax.rsqrt(rsum)
    adjmt = jnp.transpose(adjm)

    ant = pl.pallas_call(
        functools.partial(_scale_body, rblk=rblk),
        grid=(grid_r,),
        in_specs=[
            pl.BlockSpec((rblk, n), lambda i: (i, 0)),
            pl.BlockSpec((rblk, 1), lambda i: (i, 0)),
            pl.BlockSpec((1, n), lambda i: (0, 0)),
        ],
        out_specs=pl.BlockSpec((rblk, n), lambda i: (i, 0)),
        out_shape=jax.ShapeDtypeStruct((n, n), jnp.float32),
    )(adjmt, norm, norm.reshape(1, n))

    bs = 4 if b % 4 == 0 else 1
    out = pl.pallas_call(
        functools.partial(_prop_body, bs=bs),
        grid=(b // bs,),
        in_specs=[
            pl.BlockSpec((bs, w, n), lambda i: (i, 0, 0)),
            pl.BlockSpec((n, n), lambda i: (0, 0)),
            pl.BlockSpec((w1, w), lambda i: (0, 0)),
            pl.BlockSpec((w1, 1), lambda i: (0, 0)),
            pl.BlockSpec((w2, w1), lambda i: (0, 0)),
            pl.BlockSpec((w2, 1), lambda i: (0, 0)),
        ],
        out_specs=pl.BlockSpec((bs, w2, n), lambda i: (i, 0, 0)),
        out_shape=jax.ShapeDtypeStruct((b, w2, n), jnp.float32),
    )(x, ant, d1_w, d1_b.reshape(w1, 1), d2_w, d2_b.reshape(w2, 1))

    return jnp.transpose(out, (0, 2, 1))


# speedup vs baseline: 3.0900x; 1.0757x over previous
import functools

import jax
import jax.numpy as jnp
from jax.experimental import pallas as pl

ALPHA = 3.0
K = 30


def _adj_body(adj_ref, scored_ref, adjm_ref, rsum_ref):
    adj = adj_ref[...]
    scored = scored_ref[...]
    rb = adj.shape[0]
    lo = jnp.zeros((rb, 1), jnp.float32)
    hi = jnp.full((rb, 1), 1.02, jnp.float32)
    for _ in range(33):
        mid = 0.5 * (lo + hi)
        cnt = jnp.sum(jnp.where(scored > mid, 1.0, 0.0), axis=1,
                      keepdims=True)
        pred = cnt >= K
        lo = jnp.where(pred, mid, lo)
        hi = jnp.where(pred, hi, mid)
    t = hi
    gt = scored > t
    cg = jnp.sum(jnp.where(gt, 1.0, 0.0), axis=1, keepdims=True)
    ties = scored == t
    need = K - cg
    n = adj.shape[1]
    col = jax.lax.broadcasted_iota(jnp.int32, (rb, n), 1)
    clo = jnp.full((rb, 1), -1, jnp.int32)
    chi = jnp.full((rb, 1), n - 1, jnp.int32)
    for _ in range(11):
        cmid = (clo + chi) // 2
        c = jnp.sum(jnp.where(ties & (col <= cmid), 1.0, 0.0), axis=1,
                    keepdims=True)
        pred = c >= need
        chi = jnp.where(pred, cmid, chi)
        clo = jnp.where(pred, clo, cmid)
    sel = gt | (ties & (col <= chi))
    adjm = jnp.where(sel, adj, 0.0)
    adjm_ref[...] = adjm
    rsum_ref[...] = jnp.sum(adjm, axis=1, keepdims=True) + 1.0


def _scale_body(adjm_ref, normr_ref, normc_ref, ann_ref, *, rblk):
    r0 = pl.program_id(0) * rblk
    blk = adjm_ref[...]
    nrows, ncols = blk.shape
    rowi = jax.lax.broadcasted_iota(jnp.int32, (nrows, ncols), 0) + r0
    coli = jax.lax.broadcasted_iota(jnp.int32, (nrows, ncols), 1)
    a2 = blk + jnp.where(rowi == coli, 1.0, 0.0)
    ann_ref[...] = (normr_ref[...] * a2 * normc_ref[...]).astype(
        ann_ref.dtype)


def _prop_body(x_ref, ann_ref, d1w_ref, d1b_ref, d2w_ref, d2b_ref, out_ref,
               *, bs):
    ann = ann_ref[...]
    dn = (((1,), (1,)), ((), ()))
    h1s = []
    for k in range(bs):
        xk = x_ref[k]
        y1 = jnp.dot(d1w_ref[...], xk, preferred_element_type=jnp.float32)
        h1s.append(y1 + d1b_ref[...])
    h1 = jnp.concatenate(h1s, axis=0).astype(ann.dtype)
    z1 = jnp.maximum(
        jax.lax.dot_general(h1, ann, dn,
                            preferred_element_type=jnp.float32), 0.0)
    w1 = d1w_ref.shape[0]
    h2s = []
    for k in range(bs):
        z1k = z1[k * w1:(k + 1) * w1]
        y2 = jnp.dot(d2w_ref[...], z1k, preferred_element_type=jnp.float32)
        h2s.append(y2 + d2b_ref[...])
    h2 = jnp.concatenate(h2s, axis=0).astype(ann.dtype)
    z2 = jnp.maximum(
        jax.lax.dot_general(h2, ann, dn,
                            preferred_element_type=jnp.float32), 0.0)
    w2 = d2w_ref.shape[0]
    for k in range(bs):
        out_ref[k] = z2[k * w2:(k + 1) * w2]


def kernel(x, e1, e2, l1_w, l1_b, l2_w, l2_b, d1_w, d1_b, d2_w, d2_b, idx):
    n, w = e1.shape
    b = x.shape[0]
    w1 = d1_w.shape[0]
    w2 = d2_w.shape[0]

    e1g = jnp.take(e1, idx, axis=0)
    e2g = jnp.take(e2, idx, axis=0)
    noise = jax.random.uniform(jax.random.key(12345), (n, n),
                               dtype=jnp.float32) * 0.01

    m1 = jnp.tanh(ALPHA * (e1g @ l1_w.T + l1_b))
    m2 = jnp.tanh(ALPHA * (e2g @ l2_w.T + l2_b))
    adj = jax.nn.relu(jnp.tanh(ALPHA * (m1 @ m2.T)))
    scored = adj + noise

    rblk = n // 5 if n % 5 == 0 and (n // 5) % 8 == 0 else n
    grid_r = n // rblk
    adjm, rsum = pl.pallas_call(
        _adj_body,
        grid=(grid_r,),
        in_specs=[
            pl.BlockSpec((rblk, n), lambda i: (i, 0)),
            pl.BlockSpec((rblk, n), lambda i: (i, 0)),
        ],
        out_specs=(pl.BlockSpec((rblk, n), lambda i: (i, 0)),
                   pl.BlockSpec((rblk, 1), lambda i: (i, 0))),
        out_shape=(jax.ShapeDtypeStruct((n, n), jnp.float32),
                   jax.ShapeDtypeStruct((n, 1), jnp.float32)),
    )(adj, scored)

    norm = jax.lax.rsqrt(rsum)

    ann = pl.pallas_call(
        functools.partial(_scale_body, rblk=rblk),
        grid=(grid_r,),
        in_specs=[
            pl.BlockSpec((rblk, n), lambda i: (i, 0)),
            pl.BlockSpec((rblk, 1), lambda i: (i, 0)),
            pl.BlockSpec((1, n), lambda i: (0, 0)),
        ],
        out_specs=pl.BlockSpec((rblk, n), lambda i: (i, 0)),
        out_shape=jax.ShapeDtypeStruct((n, n), jnp.bfloat16),
    )(adjm, norm, norm.reshape(1, n))

    bs = 4 if b % 4 == 0 else 1
    out = pl.pallas_call(
        functools.partial(_prop_body, bs=bs),
        grid=(b // bs,),
        in_specs=[
            pl.BlockSpec((bs, w, n), lambda i: (i, 0, 0)),
            pl.BlockSpec((n, n), lambda i: (0, 0)),
            pl.BlockSpec((w1, w), lambda i: (0, 0)),
            pl.BlockSpec((w1, 1), lambda i: (0, 0)),
            pl.BlockSpec((w2, w1), lambda i: (0, 0)),
            pl.BlockSpec((w2, 1), lambda i: (0, 0)),
        ],
        out_specs=pl.BlockSpec((bs, w2, n), lambda i: (i, 0, 0)),
        out_shape=jax.ShapeDtypeStruct((b, w2, n), jnp.float32),
    )(x, ann, d1_w, d1_b.reshape(w1, 1), d2_w, d2_b.reshape(w2, 1))

    return jnp.transpose(out, (0, 2, 1))
